# speedup vs baseline: 1.2770x; 1.2770x over previous
import functools

import jax
import jax.numpy as jnp
from jax import lax
from jax.experimental import pallas as pl
from jax.experimental.pallas import tpu as pltpu
from jax.experimental.pallas import tpu_sc as plsc

BATCH = 4
SEQ = 2048
EMBED = 128
NW = 32
ROWS_PER_W = (BATCH * SEQ) // NW
LANES = 16


def _body(x_hbm, tok_hbm, pos_hbm, out_hbm, idx_v, tok_v, pos_v, sem):
    cid = lax.axis_index("c")
    sid = lax.axis_index("s")
    wid = sid * 2 + cid
    base = wid * ROWS_PER_W

    pltpu.sync_copy(x_hbm.at[pl.ds(wid * 2, 2)], idx_v)

    g0 = pltpu.async_copy(tok_hbm.at[idx_v.at[0]], tok_v.at[pl.ds(0, 128)], sem)
    g1 = pltpu.async_copy(tok_hbm.at[idx_v.at[1]], tok_v.at[pl.ds(128, 128)], sem)
    p0 = pltpu.async_copy(
        pos_hbm.at[pl.ds(lax.rem(wid, SEQ // ROWS_PER_W) * ROWS_PER_W, ROWS_PER_W)],
        pos_v, sem)
    g0.wait()
    g1.wait()
    p0.wait()

    def row(r, carry):
        for c in range(EMBED // LANES):
            sl = pl.ds(c * LANES, LANES)
            plsc.addupdate(tok_v.at[r, sl], pos_v[r, sl])
        return carry

    lax.fori_loop(0, ROWS_PER_W, row, 0)

    pltpu.sync_copy(tok_v, out_hbm.at[pl.ds(base, ROWS_PER_W)])


@jax.jit
def _emb_lookup(x2d, token_table, position_table):
    mesh = plsc.VectorSubcoreMesh(core_axis_name="c", subcore_axis_name="s")
    return pl.kernel(
        _body,
        mesh=mesh,
        out_type=jax.ShapeDtypeStruct((BATCH * SEQ, EMBED), jnp.float32),
        scratch_types=[
            pltpu.VMEM((2, 128), jnp.int32),
            pltpu.VMEM((ROWS_PER_W, EMBED), jnp.float32),
            pltpu.VMEM((ROWS_PER_W, EMBED), jnp.float32),
            pltpu.SemaphoreType.DMA,
        ],
    )(x2d, token_table, position_table)


def kernel(x, token_table, position_table):
    x2d = x.reshape(NW * 2, 128).astype(jnp.int32)
    out = _emb_lookup(x2d, token_table, position_table)
    return out.reshape(BATCH, SEQ, EMBED)


# speedup vs baseline: 1.2954x; 1.0144x over previous
import functools

import jax
import jax.numpy as jnp
from jax import lax
from jax.experimental import pallas as pl
from jax.experimental.pallas import tpu as pltpu
from jax.experimental.pallas import tpu_sc as plsc

BATCH = 4
SEQ = 2048
EMBED = 128
NW = 32
ROWS_PER_W = (BATCH * SEQ) // NW
LANES = 16


CHUNK = 128


def _body(x_hbm, tok_hbm, pos_hbm, out_hbm, idx_v, tok_v, pos_v,
          sem0, sem1, st_sem):
    cid = lax.axis_index("c")
    sid = lax.axis_index("s")
    wid = sid * 2 + cid
    base = wid * ROWS_PER_W
    s_base = lax.rem(wid, SEQ // ROWS_PER_W) * ROWS_PER_W

    pltpu.sync_copy(x_hbm.at[pl.ds(wid * 2, 2)], idx_v)

    sems = (sem0, sem1)
    loads = []
    for j in range(2):
        off = j * CHUNK
        g = pltpu.async_copy(tok_hbm.at[idx_v.at[j]],
                             tok_v.at[pl.ds(off, CHUNK)], sems[j])
        p = pltpu.async_copy(pos_hbm.at[pl.ds(s_base + off, CHUNK)],
                             pos_v.at[pl.ds(off, CHUNK)], sems[j])
        loads.append((g, p))

    def add_rows(off):
        def row(r, carry):
            for c in range(EMBED // LANES):
                sl = pl.ds(c * LANES, LANES)
                plsc.addupdate(tok_v.at[off + r, sl], pos_v[off + r, sl])
            return carry
        lax.fori_loop(0, CHUNK, row, 0)

    stores = []
    for j in range(2):
        off = j * CHUNK
        g, p = loads[j]
        g.wait()
        p.wait()
        add_rows(off)
        stores.append(pltpu.async_copy(tok_v.at[pl.ds(off, CHUNK)],
                                       out_hbm.at[pl.ds(base + off, CHUNK)],
                                       st_sem))
    for st in stores:
        st.wait()


@jax.jit
def _emb_lookup(x2d, token_table, position_table):
    mesh = plsc.VectorSubcoreMesh(core_axis_name="c", subcore_axis_name="s")
    return pl.kernel(
        _body,
        mesh=mesh,
        out_type=jax.ShapeDtypeStruct((BATCH * SEQ, EMBED), jnp.float32),
        scratch_types=[
            pltpu.VMEM((2, 128), jnp.int32),
            pltpu.VMEM((ROWS_PER_W, EMBED), jnp.float32),
            pltpu.VMEM((ROWS_PER_W, EMBED), jnp.float32),
            pltpu.SemaphoreType.DMA,
            pltpu.SemaphoreType.DMA,
            pltpu.SemaphoreType.DMA,
        ],
    )(x2d, token_table, position_table)


def kernel(x, token_table, position_table):
    x2d = x.reshape(NW * 2, 128).astype(jnp.int32)
    out = _emb_lookup(x2d, token_table, position_table)
    return out.reshape(BATCH, SEQ, EMBED)


# speedup vs baseline: 1.3077x; 1.0095x over previous
import functools

import jax
import jax.numpy as jnp
from jax import lax
from jax.experimental import pallas as pl
from jax.experimental.pallas import tpu as pltpu
from jax.experimental.pallas import tpu_sc as plsc

BATCH = 4
SEQ = 2048
EMBED = 128
NW = 32
ROWS_PER_W = (BATCH * SEQ) // NW
LANES = 16


NCHUNK = 4
CHUNK = ROWS_PER_W // NCHUNK


def _body(x_hbm, tok_hbm, pos_hbm, out_hbm, idx_v, tok_v, pos_v,
          g0, g1, g2, g3, p_sem, st_sem):
    cid = lax.axis_index("c")
    sid = lax.axis_index("s")
    wid = sid * 2 + cid
    base = wid * ROWS_PER_W
    s_base = lax.rem(wid, SEQ // ROWS_PER_W) * ROWS_PER_W

    pos_cp = pltpu.async_copy(pos_hbm.at[pl.ds(s_base, ROWS_PER_W)], pos_v,
                              p_sem)
    pltpu.sync_copy(x_hbm.at[pl.ds(wid * 2, 2)], idx_v)

    g_sems = (g0, g1, g2, g3)
    gathers = []
    for j in range(NCHUNK):
        idx_sl = idx_v.at[(j * CHUNK) // 128, pl.ds((j * CHUNK) % 128, CHUNK)]
        gathers.append(pltpu.async_copy(
            tok_hbm.at[idx_sl], tok_v.at[pl.ds(j * CHUNK, CHUNK)], g_sems[j]))

    pos_cp.wait()
    stores = []
    for j in range(NCHUNK):
        off = j * CHUNK
        gathers[j].wait()

        @plsc.parallel_loop(0, CHUNK, unroll=2)
        def row(r):
            for c in range(EMBED // LANES):
                sl = pl.ds(c * LANES, LANES)
                plsc.addupdate(tok_v.at[off + r, sl], pos_v[off + r, sl])

        stores.append(pltpu.async_copy(tok_v.at[pl.ds(off, CHUNK)],
                                       out_hbm.at[pl.ds(base + off, CHUNK)],
                                       st_sem))
    for st in stores:
        st.wait()


@jax.jit
def _emb_lookup(x2d, token_table, position_table):
    mesh = plsc.VectorSubcoreMesh(core_axis_name="c", subcore_axis_name="s")
    return pl.kernel(
        _body,
        mesh=mesh,
        out_type=jax.ShapeDtypeStruct((BATCH * SEQ, EMBED), jnp.float32),
        scratch_types=[
            pltpu.VMEM((2, 128), jnp.int32),
            pltpu.VMEM((ROWS_PER_W, EMBED), jnp.float32),
            pltpu.VMEM((ROWS_PER_W, EMBED), jnp.float32),
            pltpu.SemaphoreType.DMA,
            pltpu.SemaphoreType.DMA,
            pltpu.SemaphoreType.DMA,
            pltpu.SemaphoreType.DMA,
            pltpu.SemaphoreType.DMA,
            pltpu.SemaphoreType.DMA,
        ],
    )(x2d, token_table, position_table)


def kernel(x, token_table, position_table):
    x2d = x.reshape(NW * 2, 128).astype(jnp.int32)
    out = _emb_lookup(x2d, token_table, position_table)
    return out.reshape(BATCH, SEQ, EMBED)


# speedup vs baseline: 1.3488x; 1.0314x over previous
import functools

import jax
import jax.numpy as jnp
from jax import lax
from jax.experimental import pallas as pl
from jax.experimental.pallas import tpu as pltpu
from jax.experimental.pallas import tpu_sc as plsc

BATCH = 4
SEQ = 2048
EMBED = 128
NW = 32
ROWS_PER_W = (BATCH * SEQ) // NW
LANES = 16


SPW = SEQ // NW


def _body(x_hbm, tok_hbm, pos_hbm, out_hbm, idx_v, tok_v, pos_v,
          g0, g1, g2, g3, p_sem, st_sem):
    cid = lax.axis_index("c")
    sid = lax.axis_index("s")
    wid = sid * 2 + cid
    s0 = wid * SPW

    pos_cp = pltpu.async_copy(pos_hbm.at[pl.ds(s0, SPW)], pos_v, p_sem)
    idx_cps = [pltpu.async_copy(x_hbm.at[b, pl.ds(s0, SPW)], idx_v.at[b],
                                st_sem) for b in range(BATCH)]
    for cp in idx_cps:
        cp.wait()

    g_sems = (g0, g1, g2, g3)
    gathers = []
    for b in range(BATCH):
        gathers.append(pltpu.async_copy(
            tok_hbm.at[idx_v.at[b]], tok_v.at[pl.ds(b * SPW, SPW)],
            g_sems[b]))

    pos_cp.wait()
    stores = []
    for b in range(BATCH):
        off = b * SPW
        gathers[b].wait()

        @plsc.parallel_loop(0, SPW, unroll=2)
        def row(r):
            for c in range(EMBED // LANES):
                sl = pl.ds(c * LANES, LANES)
                plsc.addupdate(tok_v.at[off + r, sl], pos_v[r, sl])

        stores.append(pltpu.async_copy(
            tok_v.at[pl.ds(off, SPW)],
            out_hbm.at[pl.ds(b * SEQ + s0, SPW)], st_sem))
    for st in stores:
        st.wait()


@jax.jit
def _emb_lookup(x2d, token_table, position_table):
    mesh = plsc.VectorSubcoreMesh(core_axis_name="c", subcore_axis_name="s")
    return pl.kernel(
        _body,
        mesh=mesh,
        out_type=jax.ShapeDtypeStruct((BATCH * SEQ, EMBED), jnp.float32),
        scratch_types=[
            pltpu.VMEM((BATCH, SPW), jnp.int32),
            pltpu.VMEM((ROWS_PER_W, EMBED), jnp.float32),
            pltpu.VMEM((SPW, EMBED), jnp.float32),
            pltpu.SemaphoreType.DMA,
            pltpu.SemaphoreType.DMA,
            pltpu.SemaphoreType.DMA,
            pltpu.SemaphoreType.DMA,
            pltpu.SemaphoreType.DMA,
            pltpu.SemaphoreType.DMA,
        ],
    )(x2d, token_table, position_table)


def kernel(x, token_table, position_table):
    out = _emb_lookup(x.astype(jnp.int32), token_table, position_table)
    return out.reshape(BATCH, SEQ, EMBED)
